# baseline (speedup 1.0000x reference)
import functools

import jax
import jax.numpy as jnp
from jax import lax
from jax.experimental import pallas as pl
from jax.experimental.pallas import tpu as pltpu
from jax.experimental.pallas import tpu_sc as plsc

T, D, E, K = 2048, 768, 8, 2
H = 4 * D
A = T * K
NB = 256
G_MAX = A // NB + E
A_PAD = G_MAX * NB

NC, NS, L = 2, 16, 16
NW = NC * NS


@functools.cache
def _mesh():
    return plsc.VectorSubcoreMesh(core_axis_name="c", subcore_axis_name="s")


_SC_PARAMS = pltpu.CompilerParams(needs_layout_passes=False)


def _router_body(x_ref, n_ref, wg_ref, bg_ref, wn_ref, bn_ref,
                 e_ref, g_ref):
    xb = x_ref[0]
    logits = jnp.dot(xb, wg_ref[...],
                     preferred_element_type=jnp.float32) + bg_ref[...]
    nlog = jnp.dot(xb, wn_ref[...],
                   preferred_element_type=jnp.float32) + bn_ref[...]
    sp = jnp.maximum(nlog, 0.0) + jnp.log1p(jnp.exp(-jnp.abs(nlog)))
    noisy = logits + n_ref[...] * sp
    lane = lax.broadcasted_iota(jnp.int32, (T, E), 1)
    top1 = jnp.max(noisy, axis=1, keepdims=True)
    idx1 = jnp.min(jnp.where(noisy == top1, lane, E), axis=1, keepdims=True)
    noisy2 = jnp.where(lane == idx1, -jnp.inf, noisy)
    top2 = jnp.max(noisy2, axis=1, keepdims=True)
    idx2 = jnp.min(jnp.where(noisy2 == top2, lane, E), axis=1, keepdims=True)
    sel = (lane == idx1) | (lane == idx2)
    p = jnp.where(sel, jnp.exp(noisy - top1), 0.0)
    z = jnp.sum(p, axis=1, keepdims=True)
    p1 = jnp.sum(jnp.where(lane == idx1, p, 0.0), axis=1, keepdims=True)
    p2 = jnp.sum(jnp.where(lane == idx2, p, 0.0), axis=1, keepdims=True)
    e_ref[0:T, :] = idx1
    e_ref[T:2 * T, :] = idx2
    g_ref[0:T, :] = p1 / z
    g_ref[T:2 * T, :] = p2 / z


def _router(x, nf, Wg, bg, Wn, bn):
    full = lambda s: pl.BlockSpec(s, lambda: (0,) * len(s))
    return pl.pallas_call(
        _router_body,
        in_specs=[full((1, T, D)), full((T, E)), full((D, E)), full((1, E)),
                  full((D, E)), full((1, E))],
        out_specs=[full((A, 1)), full((A, 1))],
        out_shape=[jax.ShapeDtypeStruct((A, 1), jnp.int32),
                   jax.ShapeDtypeStruct((A, 1), jnp.float32)],
        name="tc_router",
    )(x, nf, Wg, bg.reshape(1, E), Wn, bn.reshape(1, E))


CPW = A // NS
VPW = CPW // L


_G_ROWS = A_PAD // NW
_G_CHUNK = 64


def _dispatch_body(e_hbm, g_hbm, x_hbm,
                   xg_hbm, gate_hbm, bexp_hbm, dest_hbm, nblk_hbm,
                   toks_sh, ev, gv, rank, destv, toks, gts, offs_v, bexp_v,
                   nblk_v, idx_v, r0, r1, sg0, sg1, sw0, sw1):
    core = lax.axis_index("c")
    sub = lax.axis_index("s")

    @pl.when(sub == 0)
    def _():
        iota = lax.iota(jnp.int32, L)
        zz = jnp.zeros((L,), jnp.int32)
        one = jnp.ones((L,), jnp.int32)
        pltpu.sync_copy(e_hbm, ev)
        pltpu.sync_copy(g_hbm, gv)

        def p1(i, cnts):
            v = ev[pl.ds(i * L, L)]
            r = zz
            new = []
            for ex in range(E):
                m = v == jnp.full((L,), ex, jnp.int32)
                mi = jnp.where(m, one, zz)
                pc = plsc.cumsum(mi)
                r = jnp.where(m, cnts[ex] + pc - one, r)
                new.append(cnts[ex] + jnp.full((L,), jnp.sum(mi), jnp.int32))
            rank[pl.ds(i * L, L)] = r
            return tuple(new)
        counts = lax.fori_loop(0, A // L, p1, (zz,) * E)

        nbv = jnp.full((L,), NB, jnp.int32)
        nbm1 = jnp.full((L,), NB - 1, jnp.int32)
        cb = jnp.zeros((L,), jnp.int32)
        off_v = jnp.zeros((L,), jnp.int32)
        be0 = jnp.zeros((L,), jnp.int32)
        be1 = jnp.zeros((L,), jnp.int32)
        iota_hi = iota + jnp.full((L,), L, jnp.int32)
        for ex in range(E):
            off_v = jnp.where(iota == jnp.full((L,), ex, jnp.int32),
                              cb * nbv, off_v)
            cb = cb + (counts[ex] + nbm1) // nbv
            be0 = be0 + jnp.where(iota >= cb, one, zz)
            be1 = be1 + jnp.where(iota_hi >= cb, one, zz)
        emax = jnp.full((L,), E - 1, jnp.int32)
        offs_v[...] = off_v
        bexp_v[pl.ds(0, L)] = jnp.minimum(be0, emax)
        bexp_v[pl.ds(L, L)] = jnp.minimum(be1, emax)
        nblk_v[...] = cb

        def p2(i, _):
            v = ev[pl.ds(i * L, L)]
            r = rank[pl.ds(i * L, L)]
            g = gv[pl.ds(i * L, L)]
            dst = plsc.load_gather(offs_v, [v]) + r
            tok = (jnp.full((L,), i * L, jnp.int32) + iota) & jnp.full(
                (L,), T - 1, jnp.int32)
            plsc.store_scatter(toks, [dst], tok)
            plsc.store_scatter(gts, [dst], g)
            destv[pl.ds(i * L, L)] = dst
            return 0
        lax.fori_loop(0, A // L, p2, 0)

        pltpu.sync_copy(toks, toks_sh)

        @pl.when(core == 0)
        def _outs():
            pltpu.sync_copy(gts, gate_hbm)
            pltpu.sync_copy(destv, dest_hbm)
            pltpu.sync_copy(bexp_v, bexp_hbm)
            pltpu.sync_copy(nblk_v, nblk_hbm)

    plsc.subcore_barrier()

    wid = sub * NC + core
    base = wid * _G_ROWS
    tmask = jnp.full((L,), T - 1, jnp.int32)
    pltpu.sync_copy(toks_sh.at[pl.ds(base, _G_ROWS)], idx_v)
    for j in range(_G_ROWS // L):
        s = pl.ds(j * L, L)
        idx_v[s] = idx_v[s] & tmask
    g0 = pltpu.async_copy(x_hbm.at[idx_v.at[pl.ds(0, _G_CHUNK)]], r0, sg0)
    g1 = pltpu.async_copy(
        x_hbm.at[idx_v.at[pl.ds(_G_CHUNK, _G_CHUNK)]], r1, sg1)
    g0.wait()
    w0 = pltpu.async_copy(r0, xg_hbm.at[pl.ds(base, _G_CHUNK)], sw0)
    g1.wait()
    w1 = pltpu.async_copy(
        r1, xg_hbm.at[pl.ds(base + _G_CHUNK, _G_CHUNK)], sw1)
    w0.wait()
    g2 = pltpu.async_copy(
        x_hbm.at[idx_v.at[pl.ds(2 * _G_CHUNK, _G_CHUNK)]], r0, sg0)
    g2.wait()
    w2 = pltpu.async_copy(
        r0, xg_hbm.at[pl.ds(base + 2 * _G_CHUNK, _G_CHUNK)], sw0)
    w1.wait()
    w2.wait()


def _dispatch(*args):
    return pl.kernel(
        _dispatch_body,
        mesh=_mesh(),
        compiler_params=_SC_PARAMS,
        out_type=[jax.ShapeDtypeStruct((A_PAD, D), jnp.float32),
                  jax.ShapeDtypeStruct((A_PAD,), jnp.float32),
                  jax.ShapeDtypeStruct((2 * L,), jnp.int32),
                  jax.ShapeDtypeStruct((A,), jnp.int32),
                  jax.ShapeDtypeStruct((L,), jnp.int32)],
        scratch_types=[pltpu.VMEM_SHARED((A_PAD,), jnp.int32),
                       pltpu.VMEM((A,), jnp.int32),
                       pltpu.VMEM((A,), jnp.float32),
                       pltpu.VMEM((A,), jnp.int32),
                       pltpu.VMEM((A,), jnp.int32),
                       pltpu.VMEM((A_PAD,), jnp.int32),
                       pltpu.VMEM((A_PAD,), jnp.float32),
                       pltpu.VMEM((L,), jnp.int32),
                       pltpu.VMEM((2 * L,), jnp.int32),
                       pltpu.VMEM((L,), jnp.int32),
                       pltpu.VMEM((_G_ROWS,), jnp.int32),
                       pltpu.VMEM((_G_CHUNK, D), jnp.float32),
                       pltpu.VMEM((_G_CHUNK, D), jnp.float32),
                       pltpu.SemaphoreType.DMA,
                       pltpu.SemaphoreType.DMA,
                       pltpu.SemaphoreType.DMA,
                       pltpu.SemaphoreType.DMA],
        name="sc_dispatch",
    )(*args)


HC = 3
HB = H // HC


def _ffn_body(bexp_ref, nblk_ref, xg_ref, gate_ref, w1_ref, b1_ref, w2_ref,
              b2_ref, out_ref):
    hc = pl.program_id(1)

    @pl.when(pl.program_id(0) < nblk_ref[0])
    def _():
        xb = xg_ref[...].astype(jnp.bfloat16)
        w1 = w1_ref[0].astype(jnp.bfloat16)
        h = jnp.dot(xb, w1, preferred_element_type=jnp.float32) + b1_ref[0]
        h = jnp.maximum(h, 0.0).astype(jnp.bfloat16)
        w2 = w2_ref[0].astype(jnp.bfloat16)
        o = jnp.dot(h, w2, preferred_element_type=jnp.float32)

        @pl.when(hc == 0)
        def _first():
            out_ref[...] = (o + b2_ref[0]) * gate_ref[...]

        @pl.when(hc > 0)
        def _rest():
            out_ref[...] = out_ref[...] + o * gate_ref[...]


def _ffn(bexp, nblk, xg, gates, W1, b1, W2, b2):
    return pl.pallas_call(
        _ffn_body,
        grid_spec=pltpu.PrefetchScalarGridSpec(
            num_scalar_prefetch=2,
            grid=(G_MAX, HC),
            in_specs=[
                pl.BlockSpec((NB, D), lambda g, hc, be, nb: (g, 0)),
                pl.BlockSpec((NB, 1), lambda g, hc, be, nb: (g, 0)),
                pl.BlockSpec((1, D, HB), lambda g, hc, be, nb: (be[g], 0, hc)),
                pl.BlockSpec((1, 1, HB), lambda g, hc, be, nb: (be[g], 0, hc)),
                pl.BlockSpec((1, HB, D), lambda g, hc, be, nb: (be[g], hc, 0)),
                pl.BlockSpec((1, 1, D), lambda g, hc, be, nb: (be[g], 0, 0)),
            ],
            out_specs=pl.BlockSpec((NB, D), lambda g, hc, be, nb: (g, 0)),
        ),
        out_shape=jax.ShapeDtypeStruct((A_PAD, D), jnp.float32),
        compiler_params=pltpu.CompilerParams(
            vmem_limit_bytes=100 * 1024 * 1024),
        name="tc_ffn",
    )(bexp, nblk, xg, gates, W1, b1.reshape(E, 1, H), W2,
      b2.reshape(E, 1, D))


_C_CHUNK = T // NW


def _combine_body(os_hbm, dest_hbm, fin_hbm, d0_v, d1_v, acc_v, row_v,
                  s0, s1):
    wid = lax.axis_index("s") * NC + lax.axis_index("c")
    base = wid * _C_CHUNK
    pltpu.sync_copy(dest_hbm.at[pl.ds(base, _C_CHUNK)], d0_v)
    pltpu.sync_copy(dest_hbm.at[pl.ds(T + base, _C_CHUNK)], d1_v)
    g0 = pltpu.async_copy(os_hbm.at[d0_v], acc_v, s0)
    g1 = pltpu.async_copy(os_hbm.at[d1_v], row_v, s1)
    g0.wait()
    g1.wait()

    def add(i, _):
        for j in range(D // L):
            s = pl.ds(j * L, L)
            acc_v[i, s] = acc_v[i, s] + row_v[i, s]
        return 0
    lax.fori_loop(0, _C_CHUNK, add, 0)
    pltpu.sync_copy(acc_v, fin_hbm.at[pl.ds(base, _C_CHUNK)])


def _combine(*args):
    return pl.kernel(
        _combine_body,
        mesh=_mesh(),
        compiler_params=_SC_PARAMS,
        out_type=jax.ShapeDtypeStruct((T, D), jnp.float32),
        scratch_types=[pltpu.VMEM((_C_CHUNK,), jnp.int32),
                       pltpu.VMEM((_C_CHUNK,), jnp.int32),
                       pltpu.VMEM((_C_CHUNK, D), jnp.float32),
                       pltpu.VMEM((_C_CHUNK, D), jnp.float32),
                       pltpu.SemaphoreType.DMA,
                       pltpu.SemaphoreType.DMA],
        name="sc_combine",
    )(*args)


@jax.jit
def kernel(x, noise, Wg, bg, Wn, bn, W1, b1, W2, b2):
    nf = noise.reshape(T, E)
    e_all, g_all = _router(x, nf, Wg, bg, Wn, bn)
    xg, gate_sorted, bexp, dest, nblk = _dispatch(
        e_all.reshape(A), g_all.reshape(A), x.reshape(T, D))
    out_sorted = _ffn(bexp[:G_MAX], nblk[:1], xg,
                      gate_sorted.reshape(A_PAD, 1), W1, b1, W2, b2)
    final = _combine(out_sorted, dest)
    return final.reshape(1, T, D)


# speedup vs baseline: 1.3276x; 1.3276x over previous
import functools

import jax
import jax.numpy as jnp
from jax import lax
from jax.experimental import pallas as pl
from jax.experimental.pallas import tpu as pltpu
from jax.experimental.pallas import tpu_sc as plsc

T, D, E, K = 2048, 768, 8, 2
H = 4 * D
A = T * K
NB = 256
G_MAX = A // NB + E
A_PAD = G_MAX * NB

NC, NS, L = 2, 16, 16
NW = NC * NS


@functools.cache
def _mesh():
    return plsc.VectorSubcoreMesh(core_axis_name="c", subcore_axis_name="s")


_SC_PARAMS = pltpu.CompilerParams(needs_layout_passes=False)


def _router_body(x_ref, n_ref, wg_ref, bg_ref, wn_ref, bn_ref,
                 e_ref, g_ref):
    xb = x_ref[0]
    logits = jnp.dot(xb, wg_ref[...],
                     preferred_element_type=jnp.float32) + bg_ref[...]
    nlog = jnp.dot(xb, wn_ref[...],
                   preferred_element_type=jnp.float32) + bn_ref[...]
    sp = jnp.maximum(nlog, 0.0) + jnp.log1p(jnp.exp(-jnp.abs(nlog)))
    noisy = logits + n_ref[...] * sp
    lane = lax.broadcasted_iota(jnp.int32, (T, E), 1)
    top1 = jnp.max(noisy, axis=1, keepdims=True)
    idx1 = jnp.min(jnp.where(noisy == top1, lane, E), axis=1, keepdims=True)
    noisy2 = jnp.where(lane == idx1, -jnp.inf, noisy)
    top2 = jnp.max(noisy2, axis=1, keepdims=True)
    idx2 = jnp.min(jnp.where(noisy2 == top2, lane, E), axis=1, keepdims=True)
    sel = (lane == idx1) | (lane == idx2)
    p = jnp.where(sel, jnp.exp(noisy - top1), 0.0)
    z = jnp.sum(p, axis=1, keepdims=True)
    p1 = jnp.sum(jnp.where(lane == idx1, p, 0.0), axis=1, keepdims=True)
    p2 = jnp.sum(jnp.where(lane == idx2, p, 0.0), axis=1, keepdims=True)
    e_ref[0:T, :] = idx1
    e_ref[T:2 * T, :] = idx2
    g_ref[0:T, :] = p1 / z
    g_ref[T:2 * T, :] = p2 / z


def _router(x, nf, Wg, bg, Wn, bn):
    full = lambda s: pl.BlockSpec(s, lambda: (0,) * len(s))
    return pl.pallas_call(
        _router_body,
        in_specs=[full((1, T, D)), full((T, E)), full((D, E)), full((1, E)),
                  full((D, E)), full((1, E))],
        out_specs=[full((A, 1)), full((A, 1))],
        out_shape=[jax.ShapeDtypeStruct((A, 1), jnp.int32),
                   jax.ShapeDtypeStruct((A, 1), jnp.float32)],
        name="tc_router",
    )(x, nf, Wg, bg.reshape(1, E), Wn, bn.reshape(1, E))


CPW = A // NS
VPW = CPW // L


_G_ROWS = A_PAD // NW
_G_CHUNK = 64


def _dispatch_body(e_hbm, g_hbm, x_hbm,
                   xg_hbm, gate_hbm, bexp_hbm, dest_hbm, nblk_hbm,
                   toks_sh, ev, gv, rank, destv, toks, gts, offs_v, bexp_v,
                   nblk_v, idx_v, r0, r1, sg0, sg1, sw0, sw1):
    core = lax.axis_index("c")
    sub = lax.axis_index("s")

    @pl.when(sub == 0)
    def _():
        iota = lax.iota(jnp.int32, L)
        zz = jnp.zeros((L,), jnp.int32)
        one = jnp.ones((L,), jnp.int32)
        pltpu.sync_copy(e_hbm, ev)
        pltpu.sync_copy(g_hbm, gv)

        def p1(i, cnts):
            v = ev[pl.ds(i * L, L)]
            r = zz
            new = []
            for ex in range(E):
                m = v == jnp.full((L,), ex, jnp.int32)
                mi = jnp.where(m, one, zz)
                pc = plsc.cumsum(mi)
                r = jnp.where(m, cnts[ex] + pc - one, r)
                new.append(cnts[ex] + jnp.full((L,), jnp.sum(mi), jnp.int32))
            rank[pl.ds(i * L, L)] = r
            return tuple(new)
        counts = lax.fori_loop(0, A // L, p1, (zz,) * E)

        nbv = jnp.full((L,), NB, jnp.int32)
        nbm1 = jnp.full((L,), NB - 1, jnp.int32)
        cb = jnp.zeros((L,), jnp.int32)
        off_v = jnp.zeros((L,), jnp.int32)
        be0 = jnp.zeros((L,), jnp.int32)
        be1 = jnp.zeros((L,), jnp.int32)
        iota_hi = iota + jnp.full((L,), L, jnp.int32)
        for ex in range(E):
            off_v = jnp.where(iota == jnp.full((L,), ex, jnp.int32),
                              cb * nbv, off_v)
            cb = cb + (counts[ex] + nbm1) // nbv
            be0 = be0 + jnp.where(iota >= cb, one, zz)
            be1 = be1 + jnp.where(iota_hi >= cb, one, zz)
        emax = jnp.full((L,), E - 1, jnp.int32)
        offs_v[...] = off_v
        bexp_v[pl.ds(0, L)] = jnp.minimum(be0, emax)
        bexp_v[pl.ds(L, L)] = jnp.minimum(be1, emax)
        nblk_v[...] = cb

        def p2(i, _):
            v = ev[pl.ds(i * L, L)]
            r = rank[pl.ds(i * L, L)]
            g = gv[pl.ds(i * L, L)]
            dst = plsc.load_gather(offs_v, [v]) + r
            tok = (jnp.full((L,), i * L, jnp.int32) + iota) & jnp.full(
                (L,), T - 1, jnp.int32)
            plsc.store_scatter(toks, [dst], tok)
            plsc.store_scatter(gts, [dst], g)
            destv[pl.ds(i * L, L)] = dst
            return 0
        lax.fori_loop(0, A // L, p2, 0)

        pltpu.sync_copy(toks, toks_sh)

        @pl.when(core == 0)
        def _outs():
            pltpu.sync_copy(gts, gate_hbm)
            pltpu.sync_copy(destv, dest_hbm)
            pltpu.sync_copy(bexp_v, bexp_hbm)
            pltpu.sync_copy(nblk_v, nblk_hbm)

    plsc.subcore_barrier()

    wid = sub * NC + core
    base = wid * _G_ROWS
    tmask = jnp.full((L,), T - 1, jnp.int32)
    pltpu.sync_copy(toks_sh.at[pl.ds(base, _G_ROWS)], idx_v)
    for j in range(_G_ROWS // L):
        s = pl.ds(j * L, L)
        idx_v[s] = idx_v[s] & tmask
    g0 = pltpu.async_copy(x_hbm.at[idx_v.at[pl.ds(0, _G_CHUNK)]], r0, sg0)
    g1 = pltpu.async_copy(
        x_hbm.at[idx_v.at[pl.ds(_G_CHUNK, _G_CHUNK)]], r1, sg1)
    g0.wait()
    w0 = pltpu.async_copy(r0, xg_hbm.at[pl.ds(base, _G_CHUNK)], sw0)
    g1.wait()
    w1 = pltpu.async_copy(
        r1, xg_hbm.at[pl.ds(base + _G_CHUNK, _G_CHUNK)], sw1)
    w0.wait()
    g2 = pltpu.async_copy(
        x_hbm.at[idx_v.at[pl.ds(2 * _G_CHUNK, _G_CHUNK)]], r0, sg0)
    g2.wait()
    w2 = pltpu.async_copy(
        r0, xg_hbm.at[pl.ds(base + 2 * _G_CHUNK, _G_CHUNK)], sw0)
    w1.wait()
    w2.wait()


def _dispatch(*args):
    return pl.kernel(
        _dispatch_body,
        mesh=_mesh(),
        compiler_params=_SC_PARAMS,
        out_type=[jax.ShapeDtypeStruct((A_PAD, D), jnp.float32),
                  jax.ShapeDtypeStruct((A_PAD,), jnp.float32),
                  jax.ShapeDtypeStruct((2 * L,), jnp.int32),
                  jax.ShapeDtypeStruct((A,), jnp.int32),
                  jax.ShapeDtypeStruct((L,), jnp.int32)],
        scratch_types=[pltpu.VMEM_SHARED((A_PAD,), jnp.int32),
                       pltpu.VMEM((A,), jnp.int32),
                       pltpu.VMEM((A,), jnp.float32),
                       pltpu.VMEM((A,), jnp.int32),
                       pltpu.VMEM((A,), jnp.int32),
                       pltpu.VMEM((A_PAD,), jnp.int32),
                       pltpu.VMEM((A_PAD,), jnp.float32),
                       pltpu.VMEM((L,), jnp.int32),
                       pltpu.VMEM((2 * L,), jnp.int32),
                       pltpu.VMEM((L,), jnp.int32),
                       pltpu.VMEM((_G_ROWS,), jnp.int32),
                       pltpu.VMEM((_G_CHUNK, D), jnp.float32),
                       pltpu.VMEM((_G_CHUNK, D), jnp.float32),
                       pltpu.SemaphoreType.DMA,
                       pltpu.SemaphoreType.DMA,
                       pltpu.SemaphoreType.DMA,
                       pltpu.SemaphoreType.DMA],
        name="sc_dispatch",
    )(*args)


H2 = H // 2


def _ffn_body(bexp_ref, nblk_ref, xg_ref, gate_ref, w1a_ref, w1b_ref,
              b1_ref, w2a_ref, w2b_ref, b2_ref, out_ref):
    @pl.when(pl.program_id(0) < nblk_ref[0])
    def _():
        xb = xg_ref[...].astype(jnp.bfloat16)
        h1 = jnp.dot(xb, w1a_ref[0].astype(jnp.bfloat16),
                     preferred_element_type=jnp.float32) + b1_ref[0, :, 0:H2]
        h1 = jnp.maximum(h1, 0.0).astype(jnp.bfloat16)
        o = jnp.dot(h1, w2a_ref[0].astype(jnp.bfloat16),
                    preferred_element_type=jnp.float32)
        h2 = jnp.dot(xb, w1b_ref[0].astype(jnp.bfloat16),
                     preferred_element_type=jnp.float32) + b1_ref[0, :, H2:H]
        h2 = jnp.maximum(h2, 0.0).astype(jnp.bfloat16)
        o = o + jnp.dot(h2, w2b_ref[0].astype(jnp.bfloat16),
                        preferred_element_type=jnp.float32)
        out_ref[...] = (o + b2_ref[0]) * gate_ref[...]


def _ffn(bexp, nblk, xg, gates, W1, b1, W2, b2):
    return pl.pallas_call(
        _ffn_body,
        grid_spec=pltpu.PrefetchScalarGridSpec(
            num_scalar_prefetch=2,
            grid=(G_MAX,),
            in_specs=[
                pl.BlockSpec((NB, D), lambda g, be, nb: (g, 0)),
                pl.BlockSpec((NB, 1), lambda g, be, nb: (g, 0)),
                pl.BlockSpec((1, D, H2), lambda g, be, nb: (be[g], 0, 0)),
                pl.BlockSpec((1, D, H2), lambda g, be, nb: (be[g], 0, 1)),
                pl.BlockSpec((1, 1, H), lambda g, be, nb: (be[g], 0, 0)),
                pl.BlockSpec((1, H2, D), lambda g, be, nb: (be[g], 0, 0)),
                pl.BlockSpec((1, H2, D), lambda g, be, nb: (be[g], 1, 0)),
                pl.BlockSpec((1, 1, D), lambda g, be, nb: (be[g], 0, 0)),
            ],
            out_specs=pl.BlockSpec((NB, D), lambda g, be, nb: (g, 0)),
        ),
        out_shape=jax.ShapeDtypeStruct((A_PAD, D), jnp.float32),
        compiler_params=pltpu.CompilerParams(
            vmem_limit_bytes=100 * 1024 * 1024),
        name="tc_ffn",
    )(bexp, nblk, xg, gates, W1, W1, b1.reshape(E, 1, H), W2, W2,
      b2.reshape(E, 1, D))


_C_CHUNK = T // NW


def _combine_body(os_hbm, dest_hbm, fin_hbm, d0_v, d1_v, acc_v, row_v,
                  s0, s1):
    wid = lax.axis_index("s") * NC + lax.axis_index("c")
    base = wid * _C_CHUNK
    pltpu.sync_copy(dest_hbm.at[pl.ds(base, _C_CHUNK)], d0_v)
    pltpu.sync_copy(dest_hbm.at[pl.ds(T + base, _C_CHUNK)], d1_v)
    g0 = pltpu.async_copy(os_hbm.at[d0_v], acc_v, s0)
    g1 = pltpu.async_copy(os_hbm.at[d1_v], row_v, s1)
    g0.wait()
    g1.wait()

    def add(i, _):
        for j in range(D // L):
            s = pl.ds(j * L, L)
            acc_v[i, s] = acc_v[i, s] + row_v[i, s]
        return 0
    lax.fori_loop(0, _C_CHUNK, add, 0)
    pltpu.sync_copy(acc_v, fin_hbm.at[pl.ds(base, _C_CHUNK)])


def _combine(*args):
    return pl.kernel(
        _combine_body,
        mesh=_mesh(),
        compiler_params=_SC_PARAMS,
        out_type=jax.ShapeDtypeStruct((T, D), jnp.float32),
        scratch_types=[pltpu.VMEM((_C_CHUNK,), jnp.int32),
                       pltpu.VMEM((_C_CHUNK,), jnp.int32),
                       pltpu.VMEM((_C_CHUNK, D), jnp.float32),
                       pltpu.VMEM((_C_CHUNK, D), jnp.float32),
                       pltpu.SemaphoreType.DMA,
                       pltpu.SemaphoreType.DMA],
        name="sc_combine",
    )(*args)


@jax.jit
def kernel(x, noise, Wg, bg, Wn, bn, W1, b1, W2, b2):
    nf = noise.reshape(T, E)
    e_all, g_all = _router(x, nf, Wg, bg, Wn, bn)
    xg, gate_sorted, bexp, dest, nblk = _dispatch(
        e_all.reshape(A), g_all.reshape(A), x.reshape(T, D))
    out_sorted = _ffn(bexp[:G_MAX], nblk[:1], xg,
                      gate_sorted.reshape(A_PAD, 1), W1, b1, W2, b2)
    final = _combine(out_sorted, dest)
    return final.reshape(1, T, D)


# speedup vs baseline: 1.3659x; 1.0288x over previous
import functools

import jax
import jax.numpy as jnp
from jax import lax
from jax.experimental import pallas as pl
from jax.experimental.pallas import tpu as pltpu
from jax.experimental.pallas import tpu_sc as plsc

T, D, E, K = 2048, 768, 8, 2
H = 4 * D
A = T * K
NB = 256
G_MAX = A // NB + E
A_PAD = G_MAX * NB

NC, NS, L = 2, 16, 16
NW = NC * NS


@functools.cache
def _mesh():
    return plsc.VectorSubcoreMesh(core_axis_name="c", subcore_axis_name="s")


_SC_PARAMS = pltpu.CompilerParams(needs_layout_passes=False)


def _router_body(x_ref, n_ref, wg_ref, bg_ref, wn_ref, bn_ref,
                 e_ref, g_ref):
    xb = x_ref[0]
    logits = jnp.dot(xb, wg_ref[...],
                     preferred_element_type=jnp.float32) + bg_ref[...]
    nlog = jnp.dot(xb, wn_ref[...],
                   preferred_element_type=jnp.float32) + bn_ref[...]
    sp = jnp.maximum(nlog, 0.0) + jnp.log1p(jnp.exp(-jnp.abs(nlog)))
    noisy = logits + n_ref[...] * sp
    lane = lax.broadcasted_iota(jnp.int32, (T, E), 1)
    top1 = jnp.max(noisy, axis=1, keepdims=True)
    idx1 = jnp.min(jnp.where(noisy == top1, lane, E), axis=1, keepdims=True)
    noisy2 = jnp.where(lane == idx1, -jnp.inf, noisy)
    top2 = jnp.max(noisy2, axis=1, keepdims=True)
    idx2 = jnp.min(jnp.where(noisy2 == top2, lane, E), axis=1, keepdims=True)
    sel = (lane == idx1) | (lane == idx2)
    p = jnp.where(sel, jnp.exp(noisy - top1), 0.0)
    z = jnp.sum(p, axis=1, keepdims=True)
    p1 = jnp.sum(jnp.where(lane == idx1, p, 0.0), axis=1, keepdims=True)
    p2 = jnp.sum(jnp.where(lane == idx2, p, 0.0), axis=1, keepdims=True)
    e_ref[0:T, :] = idx1
    e_ref[T:2 * T, :] = idx2
    g_ref[0:T, :] = p1 / z
    g_ref[T:2 * T, :] = p2 / z


def _router(x, nf, Wg, bg, Wn, bn):
    full = lambda s: pl.BlockSpec(s, lambda: (0,) * len(s))
    return pl.pallas_call(
        _router_body,
        in_specs=[full((1, T, D)), full((T, E)), full((D, E)), full((1, E)),
                  full((D, E)), full((1, E))],
        out_specs=[full((A, 1)), full((A, 1))],
        out_shape=[jax.ShapeDtypeStruct((A, 1), jnp.int32),
                   jax.ShapeDtypeStruct((A, 1), jnp.float32)],
        name="tc_router",
    )(x, nf, Wg, bg.reshape(1, E), Wn, bn.reshape(1, E))


_G_ROWS = A_PAD // NW
_G_CHUNK = 64


def _dispatch_body(e_hbm, g_hbm, x_hbm,
                   xg_hbm, gate_hbm, bexp_hbm, dest_hbm, nblk_hbm,
                   toks_sh, ev, gv, rank, destv, toks, gts, offs_v, bexp_v,
                   nblk_v, idx_v, r0, r1, sg0, sg1, sw0, sw1):
    core = lax.axis_index("c")
    sub = lax.axis_index("s")

    @pl.when(sub == 0)
    def _():
        iota = lax.iota(jnp.int32, L)
        zz = jnp.zeros((L,), jnp.int32)
        one = jnp.ones((L,), jnp.int32)
        pltpu.sync_copy(e_hbm, ev)
        pltpu.sync_copy(g_hbm, gv)

        def p1(i, cnts):
            v = ev[pl.ds(i * L, L)]
            r = zz
            new = []
            for ex in range(E):
                m = v == jnp.full((L,), ex, jnp.int32)
                mi = jnp.where(m, one, zz)
                pc = plsc.cumsum(mi)
                r = jnp.where(m, cnts[ex] + pc - one, r)
                new.append(cnts[ex] + jnp.full((L,), jnp.sum(mi), jnp.int32))
            rank[pl.ds(i * L, L)] = r
            return tuple(new)
        counts = lax.fori_loop(0, A // L, p1, (zz,) * E)

        nbv = jnp.full((L,), NB, jnp.int32)
        nbm1 = jnp.full((L,), NB - 1, jnp.int32)
        cb = jnp.zeros((L,), jnp.int32)
        off_v = jnp.zeros((L,), jnp.int32)
        be0 = jnp.zeros((L,), jnp.int32)
        be1 = jnp.zeros((L,), jnp.int32)
        iota_hi = iota + jnp.full((L,), L, jnp.int32)
        for ex in range(E):
            off_v = jnp.where(iota == jnp.full((L,), ex, jnp.int32),
                              cb * nbv, off_v)
            cb = cb + (counts[ex] + nbm1) // nbv
            be0 = be0 + jnp.where(iota >= cb, one, zz)
            be1 = be1 + jnp.where(iota_hi >= cb, one, zz)
        emax = jnp.full((L,), E - 1, jnp.int32)
        offs_v[...] = off_v
        bexp_v[pl.ds(0, L)] = jnp.minimum(be0, emax)
        bexp_v[pl.ds(L, L)] = jnp.minimum(be1, emax)
        nblk_v[...] = cb

        def p2(i, _):
            v = ev[pl.ds(i * L, L)]
            r = rank[pl.ds(i * L, L)]
            g = gv[pl.ds(i * L, L)]
            dst = plsc.load_gather(offs_v, [v]) + r
            tok = (jnp.full((L,), i * L, jnp.int32) + iota) & jnp.full(
                (L,), T - 1, jnp.int32)
            plsc.store_scatter(toks, [dst], tok)
            plsc.store_scatter(gts, [dst], g)
            destv[pl.ds(i * L, L)] = dst
            return 0
        lax.fori_loop(0, A // L, p2, 0)

        pltpu.sync_copy(toks, toks_sh)

        @pl.when(core == 0)
        def _outs():
            pltpu.sync_copy(gts, gate_hbm)
            pltpu.sync_copy(destv, dest_hbm)
            pltpu.sync_copy(bexp_v, bexp_hbm)
            pltpu.sync_copy(nblk_v, nblk_hbm)

    plsc.subcore_barrier()

    wid = sub * NC + core
    base = wid * _G_ROWS
    tmask = jnp.full((L,), T - 1, jnp.int32)
    pltpu.sync_copy(toks_sh.at[pl.ds(base, _G_ROWS)], idx_v)
    for j in range(_G_ROWS // L):
        s = pl.ds(j * L, L)
        idx_v[s] = idx_v[s] & tmask
    g0 = pltpu.async_copy(x_hbm.at[idx_v.at[pl.ds(0, _G_CHUNK)]], r0, sg0)
    g1 = pltpu.async_copy(
        x_hbm.at[idx_v.at[pl.ds(_G_CHUNK, _G_CHUNK)]], r1, sg1)
    g0.wait()
    w0 = pltpu.async_copy(r0, xg_hbm.at[pl.ds(base, _G_CHUNK)], sw0)
    g1.wait()
    w1 = pltpu.async_copy(
        r1, xg_hbm.at[pl.ds(base + _G_CHUNK, _G_CHUNK)], sw1)
    w0.wait()
    g2 = pltpu.async_copy(
        x_hbm.at[idx_v.at[pl.ds(2 * _G_CHUNK, _G_CHUNK)]], r0, sg0)
    g2.wait()
    w2 = pltpu.async_copy(
        r0, xg_hbm.at[pl.ds(base + 2 * _G_CHUNK, _G_CHUNK)], sw0)
    w1.wait()
    w2.wait()


def _dispatch(*args):
    return pl.kernel(
        _dispatch_body,
        mesh=_mesh(),
        compiler_params=_SC_PARAMS,
        out_type=[jax.ShapeDtypeStruct((A_PAD, D), jnp.float32),
                  jax.ShapeDtypeStruct((A_PAD,), jnp.float32),
                  jax.ShapeDtypeStruct((2 * L,), jnp.int32),
                  jax.ShapeDtypeStruct((A,), jnp.int32),
                  jax.ShapeDtypeStruct((L,), jnp.int32)],
        scratch_types=[pltpu.VMEM_SHARED((A_PAD,), jnp.int32),
                       pltpu.VMEM((A,), jnp.int32),
                       pltpu.VMEM((A,), jnp.float32),
                       pltpu.VMEM((A,), jnp.int32),
                       pltpu.VMEM((A,), jnp.int32),
                       pltpu.VMEM((A_PAD,), jnp.int32),
                       pltpu.VMEM((A_PAD,), jnp.float32),
                       pltpu.VMEM((L,), jnp.int32),
                       pltpu.VMEM((2 * L,), jnp.int32),
                       pltpu.VMEM((L,), jnp.int32),
                       pltpu.VMEM((_G_ROWS,), jnp.int32),
                       pltpu.VMEM((_G_CHUNK, D), jnp.float32),
                       pltpu.VMEM((_G_CHUNK, D), jnp.float32),
                       pltpu.SemaphoreType.DMA,
                       pltpu.SemaphoreType.DMA,
                       pltpu.SemaphoreType.DMA,
                       pltpu.SemaphoreType.DMA],
        name="sc_dispatch",
    )(*args)


def _ffn_body(bexp_ref, nblk_ref, xg_ref, gate_ref, w1_ref, b1_ref, w2_ref,
              b2_ref, out_ref):
    @pl.when(pl.program_id(0) < nblk_ref[0])
    def _():
        xb = xg_ref[...].astype(jnp.bfloat16)
        w1 = w1_ref[0].astype(jnp.bfloat16)
        h = jnp.dot(xb, w1, preferred_element_type=jnp.float32) + b1_ref[0]
        h = jnp.maximum(h, 0.0).astype(jnp.bfloat16)
        w2 = w2_ref[0].astype(jnp.bfloat16)
        o = jnp.dot(h, w2, preferred_element_type=jnp.float32) + b2_ref[0]
        out_ref[...] = o * gate_ref[...]


def _ffn(bexp, nblk, xg, gates, W1, b1, W2, b2):
    return pl.pallas_call(
        _ffn_body,
        grid_spec=pltpu.PrefetchScalarGridSpec(
            num_scalar_prefetch=2,
            grid=(G_MAX,),
            in_specs=[
                pl.BlockSpec((NB, D), lambda g, be, nb: (g, 0)),
                pl.BlockSpec((NB, 1), lambda g, be, nb: (g, 0)),
                pl.BlockSpec((1, D, H), lambda g, be, nb: (be[g], 0, 0)),
                pl.BlockSpec((1, 1, H), lambda g, be, nb: (be[g], 0, 0)),
                pl.BlockSpec((1, H, D), lambda g, be, nb: (be[g], 0, 0)),
                pl.BlockSpec((1, 1, D), lambda g, be, nb: (be[g], 0, 0)),
            ],
            out_specs=pl.BlockSpec((NB, D), lambda g, be, nb: (g, 0)),
        ),
        out_shape=jax.ShapeDtypeStruct((A_PAD, D), jnp.float32),
        compiler_params=pltpu.CompilerParams(
            vmem_limit_bytes=100 * 1024 * 1024),
        name="tc_ffn",
    )(bexp, nblk, xg, gates, W1, b1.reshape(E, 1, H), W2,
      b2.reshape(E, 1, D))


_C_CHUNK = T // NW


def _combine_body(os_hbm, dest_hbm, fin_hbm, d0_v, d1_v, acc_v, row_v,
                  s0, s1):
    wid = lax.axis_index("s") * NC + lax.axis_index("c")
    base = wid * _C_CHUNK
    pltpu.sync_copy(dest_hbm.at[pl.ds(base, _C_CHUNK)], d0_v)
    pltpu.sync_copy(dest_hbm.at[pl.ds(T + base, _C_CHUNK)], d1_v)
    g0 = pltpu.async_copy(os_hbm.at[d0_v], acc_v, s0)
    g1 = pltpu.async_copy(os_hbm.at[d1_v], row_v, s1)
    g0.wait()
    g1.wait()

    def add(i, _):
        for j in range(D // L):
            s = pl.ds(j * L, L)
            acc_v[i, s] = acc_v[i, s] + row_v[i, s]
        return 0
    lax.fori_loop(0, _C_CHUNK, add, 0)
    pltpu.sync_copy(acc_v, fin_hbm.at[pl.ds(base, _C_CHUNK)])


def _combine(*args):
    return pl.kernel(
        _combine_body,
        mesh=_mesh(),
        compiler_params=_SC_PARAMS,
        out_type=jax.ShapeDtypeStruct((T, D), jnp.float32),
        scratch_types=[pltpu.VMEM((_C_CHUNK,), jnp.int32),
                       pltpu.VMEM((_C_CHUNK,), jnp.int32),
                       pltpu.VMEM((_C_CHUNK, D), jnp.float32),
                       pltpu.VMEM((_C_CHUNK, D), jnp.float32),
                       pltpu.SemaphoreType.DMA,
                       pltpu.SemaphoreType.DMA],
        name="sc_combine",
    )(*args)


@jax.jit
def kernel(x, noise, Wg, bg, Wn, bn, W1, b1, W2, b2):
    nf = noise.reshape(T, E)
    e_all, g_all = _router(x, nf, Wg, bg, Wn, bn)
    xg, gate_sorted, bexp, dest, nblk = _dispatch(
        e_all.reshape(A), g_all.reshape(A), x.reshape(T, D))
    out_sorted = _ffn(bexp[:G_MAX], nblk[:1], xg,
                      gate_sorted.reshape(A_PAD, 1), W1, b1, W2, b2)
    final = _combine(out_sorted, dest)
    return final.reshape(1, T, D)
